# baseline (speedup 1.0000x reference)
import functools

import numpy as np
import jax
import jax.numpy as jnp
from jax import lax
from jax.experimental import pallas as pl
from jax.experimental.pallas import tpu as pltpu
from jax.experimental.pallas import tpu_sc as plsc

N = 2097152
PHI = 0.95
SIGMA_X = 0.3
SIGMA_Y = 0.5


def _tc_body(w3_ref, cw_ref, cend_ref, pt_s, wnT_s, i1t_s, t1_s):
    f32 = jnp.float32
    lane = lax.broadcasted_iota(jnp.int32, (1, 128), 1)
    lane2 = lax.broadcasted_iota(jnp.int32, (128, 128), 1)
    row2 = lax.broadcasted_iota(jnp.int32, (128, 128), 0)

    rows16 = lax.broadcasted_iota(jnp.int32, (16, 128), 0)

    def ibody(i, acc):
        return acc + w3_ref[:, pl.ds(i, 1), :].reshape(16, 128)

    acc = lax.fori_loop(0, 1024, ibody, jnp.zeros((16, 128), f32))

    def gbody(g, s128):
        rowg = jnp.sum(jnp.where(rows16 == g, acc, 0.0), axis=0, keepdims=True)
        return s128 + rowg

    s128 = jnp.sum(jnp.where(rows16 == 0, acc, 0.0), axis=0, keepdims=True)
    s128 = lax.fori_loop(1, 16, gbody, s128)
    l1 = s128 + jnp.roll(s128, -1, axis=1)
    l2 = l1 + jnp.roll(l1, -2, axis=1)
    l3 = l2 + jnp.roll(l2, -4, axis=1)

    def jbody(j, s):
        return s + jnp.sum(jnp.where(lane == 8 * j, l3, 0.0))

    s_tot = lax.fori_loop(1, 16, jbody, jnp.sum(jnp.where(lane == 0, l3, 0.0)))

    def abody(t, _):
        gh = t // 8
        gl = (t % 8) * 128
        w_tile = w3_ref[gh, pl.ds(gl, 128), :]
        wnT_s[...] = jnp.transpose(w_tile) / s_tot
        c0 = wnT_s[pl.ds(0, 1), :]
        pt_s[pl.ds(t * 128, 1), :] = c0

        def kbody(k, c):
            c = c + wnT_s[pl.ds(k, 1), :]
            pt_s[pl.ds(t * 128 + k, 1), :] = c
            return c

        c_last = lax.fori_loop(1, 128, kbody, c0)
        t1_s[pl.ds(t, 1), :] = c_last
        return 0

    lax.fori_loop(0, 128, abody, 0)

    wnT_s[...] = jnp.transpose(t1_s[...])
    c0 = wnT_s[pl.ds(0, 1), :]
    i1t_s[pl.ds(0, 1), :] = c0

    def k1body(k, c):
        c = c + wnT_s[pl.ds(k, 1), :]
        i1t_s[pl.ds(k, 1), :] = c
        return c

    t2 = lax.fori_loop(1, 128, k1body, c0)

    e0 = jnp.sum(jnp.where(lane == 0, t2, 0.0))
    c2_0 = jnp.where(lane == 0, e0, jnp.zeros((1, 128), f32))

    def fold2(j, carry):
        s, c2 = carry
        s = s + jnp.sum(jnp.where(lane == j, t2, 0.0))
        c2 = jnp.where(lane == j, s, c2)
        return (s, c2)

    _, c2 = lax.fori_loop(1, 128, fold2, (e0, c2_0))
    off1 = jnp.where(lane == 0, 0.0, jnp.roll(c2, 1, axis=1))

    outer1t = i1t_s[...] + off1
    outer1n = jnp.transpose(outer1t)
    rolled = jnp.roll(outer1n, 1, axis=1)
    rows_rolled = jnp.roll(outer1n, 1, axis=0)
    lastcol = jnp.sum(jnp.where(lane2 == 127, rows_rolled, 0.0), axis=1,
                      keepdims=True)
    off0 = jnp.where(lane2 == 0, lastcol, rolled)
    off0 = jnp.where((lane2 == 0) & (row2 == 0), 0.0, off0)
    wnT_s[...] = off0

    def cbody(t, _):
        off0row = wnT_s[pl.ds(t, 1), :]
        cwt = pt_s[pl.ds(t * 128, 128), :] + off0row
        cw_ref[pl.ds(t * 128, 128), :] = jnp.transpose(cwt)
        cend_row = jnp.sum(jnp.where(row2 == 127, cwt, 0.0), axis=0,
                           keepdims=True)
        cend_ref[pl.ds(t, 1), :] = cend_row
        return 0

    lax.fori_loop(0, 128, cbody, 0)


def _tc_cdf(w3):
    return pl.pallas_call(
        _tc_body,
        out_shape=(jax.ShapeDtypeStruct((16384, 128), jnp.float32),
                   jax.ShapeDtypeStruct((128, 128), jnp.float32)),
        scratch_shapes=[pltpu.VMEM((16384, 128), jnp.float32),
                        pltpu.VMEM((128, 128), jnp.float32),
                        pltpu.VMEM((128, 128), jnp.float32),
                        pltpu.VMEM((128, 128), jnp.float32)],
    )(w3)


_NW = 32
_BQ = 512
_NB = N // _NW // _BQ
_C1 = float(np.log(np.float32(SIGMA_Y)))
_C2 = float(np.float32(0.5) * np.log(np.float32(2.0) * np.pi))


def _sc_search_body(cw2d, cend_h, u2d, xit1, noise2d, yt16,
                    xt_o, lw_o, at_o,
                    cend_v, u_v, noise_v, c_v, rows_v, at_v, at_idx_v,
                    xg_v, xt_v, lw_v, yt_v, sem_g, sem_in, sem_out):
    wid = lax.axis_index("s") * 2 + lax.axis_index("c")
    pltpu.sync_copy(cend_h, cend_v)
    pltpu.sync_copy(yt16, yt_v)
    base_row = wid * (_NB * 4)

    pltpu.async_copy(u2d.at[pl.ds(base_row, 4)], u_v.at[pl.ds(0, 4)], sem_in)
    pltpu.async_copy(noise2d.at[pl.ds(base_row, 4)],
                     noise_v.at[pl.ds(0, 4)], sem_in)

    def batch(b, _):
        h = (b % 2) * 4
        hn = 4 - h
        row0 = base_row + b * 4

        pltpu.make_async_copy(u2d.at[pl.ds(0, 4)],
                              u_v.at[pl.ds(h, 4)], sem_in).wait()
        pltpu.make_async_copy(noise2d.at[pl.ds(0, 4)],
                              noise_v.at[pl.ds(h, 4)], sem_in).wait()

        def veca(i, _):
            for k in range(4):
                v = i * 4 + k
                r = v // 8
                col = (v % 8) * 16
                u16 = u_v[h + r, pl.ds(col, 16)]
                base = jnp.zeros((16,), jnp.int32)
                for s in (8192, 4096, 2048, 1024, 512, 256, 128, 64, 32,
                          16, 8, 4, 2, 1):
                    probe = base + (s - 1)
                    vals = plsc.load_gather(cend_v, [probe])
                    base = base + jnp.where(vals < u16, s, 0)
                c_v[pl.ds(v * 16, 16)] = base
            return 0

        lax.fori_loop(0, 8, veca, 0)

        h_rows = pltpu.async_copy(cw2d.at[c_v], rows_v, sem_g)

        @pl.when(b + 1 < _NB)
        def _():
            pltpu.async_copy(u2d.at[pl.ds(row0 + 4, 4)],
                             u_v.at[pl.ds(hn, 4)], sem_in)
            pltpu.async_copy(noise2d.at[pl.ds(row0 + 4, 4)],
                             noise_v.at[pl.ds(hn, 4)], sem_in)

        h_rows.wait()

        def vecb(i, _):
            for k in range(4):
                v = i * 4 + k
                r = v // 8
                col = (v % 8) * 16
                u16 = u_v[h + r, pl.ds(col, 16)]
                q16 = v * 16 + lax.iota(jnp.int32, 16)
                pos = jnp.zeros((16,), jnp.int32)
                for s in (64, 32, 16, 8, 4, 2, 1):
                    probe = pos + (s - 1)
                    vals = plsc.load_gather(rows_v, [q16, probe])
                    pos = pos + jnp.where(vals < u16, s, 0)
                idx = c_v[pl.ds(v * 16, 16)] * 128 + pos
                at_v[h + r, pl.ds(col, 16)] = idx
                at_idx_v[pl.ds(v * 16, 16)] = idx
            return 0

        lax.fori_loop(0, 8, vecb, 0)

        pltpu.async_copy(xit1.at[at_idx_v], xg_v, sem_g).wait()

        def vecc(i, _):
            for k in range(4):
                v = i * 4 + k
                r = h + v // 8
                col = (v % 8) * 16
                xg = xg_v[pl.ds(v * 16, 16)]
                nz = noise_v[r, pl.ds(col, 16)]
                yt = yt_v[...]
                xt = PHI * xg + SIGMA_X * nz
                z = (yt - xt) * 2.0
                lw = -0.5 * (z * z) - _C1 - _C2
                xt_v[r, pl.ds(col, 16)] = xt
                lw_v[r, pl.ds(col, 16)] = lw
            return 0

        lax.fori_loop(0, 8, vecc, 0)

        @pl.when(b > 0)
        def _():
            pltpu.make_async_copy(xt_o.at[pl.ds(0, 4)],
                                  xt_v.at[pl.ds(hn, 4)], sem_out).wait()
            pltpu.make_async_copy(lw_o.at[pl.ds(0, 4)],
                                  lw_v.at[pl.ds(hn, 4)], sem_out).wait()
            pltpu.make_async_copy(at_o.at[pl.ds(0, 4)],
                                  at_v.at[pl.ds(hn, 4)], sem_out).wait()

        pltpu.async_copy(xt_v.at[pl.ds(h, 4)], xt_o.at[pl.ds(row0, 4)],
                         sem_out)
        pltpu.async_copy(lw_v.at[pl.ds(h, 4)], lw_o.at[pl.ds(row0, 4)],
                         sem_out)
        pltpu.async_copy(at_v.at[pl.ds(h, 4)], at_o.at[pl.ds(row0, 4)],
                         sem_out)
        return 0

    lax.fori_loop(0, _NB, batch, 0)

    hl = ((_NB - 1) % 2) * 4
    pltpu.make_async_copy(xt_o.at[pl.ds(0, 4)],
                          xt_v.at[pl.ds(hl, 4)], sem_out).wait()
    pltpu.make_async_copy(lw_o.at[pl.ds(0, 4)],
                          lw_v.at[pl.ds(hl, 4)], sem_out).wait()
    pltpu.make_async_copy(at_o.at[pl.ds(0, 4)],
                          at_v.at[pl.ds(hl, 4)], sem_out).wait()


@functools.cache
def _make_sc_search():
  return functools.partial(
    pl.kernel,
    mesh=plsc.VectorSubcoreMesh(core_axis_name="c", subcore_axis_name="s"),
    out_type=(jax.ShapeDtypeStruct((16384, 128), jnp.float32),
              jax.ShapeDtypeStruct((16384, 128), jnp.float32),
              jax.ShapeDtypeStruct((16384, 128), jnp.int32)),
    scratch_types=[pltpu.VMEM((16384,), jnp.float32),
                   pltpu.VMEM((8, 128), jnp.float32),
                   pltpu.VMEM((8, 128), jnp.float32),
                   pltpu.VMEM((512,), jnp.int32),
                   pltpu.VMEM((512, 128), jnp.float32),
                   pltpu.VMEM((8, 128), jnp.int32),
                   pltpu.VMEM((512,), jnp.int32),
                   pltpu.VMEM((512,), jnp.float32),
                   pltpu.VMEM((8, 128), jnp.float32),
                   pltpu.VMEM((8, 128), jnp.float32),
                   pltpu.VMEM((16,), jnp.float32),
                   pltpu.SemaphoreType.DMA,
                   pltpu.SemaphoreType.DMA,
                   pltpu.SemaphoreType.DMA],
    compiler_params=pltpu.CompilerParams(needs_layout_passes=False),
  )(_sc_search_body)


def kernel(xit_1, wt_1, yt, noise, uniforms):
    w3 = wt_1.reshape(16, 1024, 128)
    cw2d, cend2d = _tc_cdf(w3)
    xt2d, lw2d, at2d = _make_sc_search()(
        cw2d, cend2d.reshape(16384), uniforms.reshape(16384, 128),
        xit_1.reshape(N), noise.reshape(16384, 128),
        jnp.broadcast_to(yt.reshape(1), (16,)))
    return (xt2d.reshape(N, 1), lw2d.reshape(N, 1), at2d.reshape(N))


# speedup vs baseline: 1.1280x; 1.1280x over previous
import functools

import numpy as np
import jax
import jax.numpy as jnp
from jax import lax
from jax.experimental import pallas as pl
from jax.experimental.pallas import tpu as pltpu
from jax.experimental.pallas import tpu_sc as plsc

N = 2097152
PHI = 0.95
SIGMA_X = 0.3
SIGMA_Y = 0.5


def _tc_body(w3_ref, cw_ref, cw32_ref, pt_s, wnT_s, i1t_s, t1_s):
    f32 = jnp.float32
    lane = lax.broadcasted_iota(jnp.int32, (1, 128), 1)
    lane2 = lax.broadcasted_iota(jnp.int32, (128, 128), 1)
    row2 = lax.broadcasted_iota(jnp.int32, (128, 128), 0)

    rows16 = lax.broadcasted_iota(jnp.int32, (16, 128), 0)

    def ibody(i, acc):
        return acc + w3_ref[:, pl.ds(i, 1), :].reshape(16, 128)

    acc = lax.fori_loop(0, 1024, ibody, jnp.zeros((16, 128), f32))

    def gbody(g, s128):
        rowg = jnp.sum(jnp.where(rows16 == g, acc, 0.0), axis=0, keepdims=True)
        return s128 + rowg

    s128 = jnp.sum(jnp.where(rows16 == 0, acc, 0.0), axis=0, keepdims=True)
    s128 = lax.fori_loop(1, 16, gbody, s128)
    l1 = s128 + jnp.roll(s128, -1, axis=1)
    l2 = l1 + jnp.roll(l1, -2, axis=1)
    l3 = l2 + jnp.roll(l2, -4, axis=1)

    def jbody(j, s):
        return s + jnp.sum(jnp.where(lane == 8 * j, l3, 0.0))

    s_tot = lax.fori_loop(1, 16, jbody, jnp.sum(jnp.where(lane == 0, l3, 0.0)))

    def abody(t, _):
        gh = t // 8
        gl = (t % 8) * 128
        w_tile = w3_ref[gh, pl.ds(gl, 128), :]
        wnT_s[...] = jnp.transpose(w_tile) / s_tot
        c0 = wnT_s[pl.ds(0, 1), :]
        pt_s[pl.ds(t * 128, 1), :] = c0

        def kbody(k, c):
            c = c + wnT_s[pl.ds(k, 1), :]
            pt_s[pl.ds(t * 128 + k, 1), :] = c
            return c

        c_last = lax.fori_loop(1, 128, kbody, c0)
        t1_s[pl.ds(t, 1), :] = c_last
        return 0

    lax.fori_loop(0, 128, abody, 0)

    wnT_s[...] = jnp.transpose(t1_s[...])
    c0 = wnT_s[pl.ds(0, 1), :]
    i1t_s[pl.ds(0, 1), :] = c0

    def k1body(k, c):
        c = c + wnT_s[pl.ds(k, 1), :]
        i1t_s[pl.ds(k, 1), :] = c
        return c

    t2 = lax.fori_loop(1, 128, k1body, c0)

    e0 = jnp.sum(jnp.where(lane == 0, t2, 0.0))
    c2_0 = jnp.where(lane == 0, e0, jnp.zeros((1, 128), f32))

    def fold2(j, carry):
        s, c2 = carry
        s = s + jnp.sum(jnp.where(lane == j, t2, 0.0))
        c2 = jnp.where(lane == j, s, c2)
        return (s, c2)

    _, c2 = lax.fori_loop(1, 128, fold2, (e0, c2_0))
    off1 = jnp.where(lane == 0, 0.0, jnp.roll(c2, 1, axis=1))

    outer1t = i1t_s[...] + off1
    outer1n = jnp.transpose(outer1t)
    rolled = jnp.roll(outer1n, 1, axis=1)
    rows_rolled = jnp.roll(outer1n, 1, axis=0)
    lastcol = jnp.sum(jnp.where(lane2 == 127, rows_rolled, 0.0), axis=1,
                      keepdims=True)
    off0 = jnp.where(lane2 == 0, lastcol, rolled)
    off0 = jnp.where((lane2 == 0) & (row2 == 0), 0.0, off0)
    wnT_s[...] = off0

    def cbody(t, _):
        off0row = wnT_s[pl.ds(t, 1), :]
        cwt = pt_s[pl.ds(t * 128, 128), :] + off0row
        cw_ref[pl.ds(t * 128, 128), :] = jnp.transpose(cwt)
        for q in range(4):
            rowq = jnp.sum(jnp.where(row2 == 32 * q + 31, cwt, 0.0), axis=0,
                           keepdims=True)
            cw32_ref[pl.ds(t, 1), pl.ds(q, 1), :] = rowq.reshape(1, 1, 128)
        return 0

    lax.fori_loop(0, 128, cbody, 0)


def _tc_cdf(w3):
    return pl.pallas_call(
        _tc_body,
        out_shape=(jax.ShapeDtypeStruct((16384, 128), jnp.float32),
                   jax.ShapeDtypeStruct((128, 4, 128), jnp.float32)),
        scratch_shapes=[pltpu.VMEM((16384, 128), jnp.float32),
                        pltpu.VMEM((128, 128), jnp.float32),
                        pltpu.VMEM((128, 128), jnp.float32),
                        pltpu.VMEM((128, 128), jnp.float32)],
    )(w3)


_NW = 32
_BQ = 512
_NB = N // _NW // _BQ
_C1 = float(np.log(np.float32(SIGMA_Y)))
_C2 = float(np.float32(0.5) * np.log(np.float32(2.0) * np.pi))


def _sc_search_body(cwseg, cw32_h, u2d, xit1, noise2d, yt16,
                    xt_o, lw_o, at_o,
                    cw32_v, u_v, noise_v, c_v, rows_v, at_v, at_idx_v,
                    xg_v, xt_v, lw_v, yt_v, sem_g, sem_in, sem_out):
    wid = lax.axis_index("s") * 2 + lax.axis_index("c")
    pltpu.sync_copy(cw32_h, cw32_v)
    pltpu.sync_copy(yt16, yt_v)
    base_row = wid * (_NB * 4)

    pltpu.async_copy(u2d.at[pl.ds(base_row, 4)], u_v.at[pl.ds(0, 4)], sem_in)
    pltpu.async_copy(noise2d.at[pl.ds(base_row, 4)],
                     noise_v.at[pl.ds(0, 4)], sem_in)

    def batch(b, _):
        h = (b % 2) * 4
        hn = 4 - h
        row0 = base_row + b * 4

        pltpu.make_async_copy(u2d.at[pl.ds(0, 4)],
                              u_v.at[pl.ds(h, 4)], sem_in).wait()
        pltpu.make_async_copy(noise2d.at[pl.ds(0, 4)],
                              noise_v.at[pl.ds(h, 4)], sem_in).wait()

        def veca(i, _):
            for k in range(4):
                v = i * 4 + k
                r = v // 8
                col = (v % 8) * 16
                u16 = u_v[h + r, pl.ds(col, 16)]
                base = jnp.zeros((16,), jnp.int32)
                for s in (32768, 16384, 8192, 4096, 2048, 1024, 512, 256,
                          128, 64, 32, 16, 8, 4, 2, 1):
                    probe = base + (s - 1)
                    t16 = lax.shift_right_logical(probe, 9)
                    q16 = probe & 3
                    cc16 = lax.shift_right_logical(probe, 2) & 127
                    vals = plsc.load_gather(cw32_v, [t16, q16, cc16])
                    base = base + jnp.where(vals < u16, s, 0)
                c_v[pl.ds(v * 16, 16)] = base
            return 0

        lax.fori_loop(0, 8, veca, 0)

        h_rows = pltpu.async_copy(cwseg.at[c_v], rows_v, sem_g)

        @pl.when(b + 1 < _NB)
        def _():
            pltpu.async_copy(u2d.at[pl.ds(row0 + 4, 4)],
                             u_v.at[pl.ds(hn, 4)], sem_in)
            pltpu.async_copy(noise2d.at[pl.ds(row0 + 4, 4)],
                             noise_v.at[pl.ds(hn, 4)], sem_in)

        h_rows.wait()

        def vecb(i, _):
            for k in range(4):
                v = i * 4 + k
                r = v // 8
                col = (v % 8) * 16
                u16 = u_v[h + r, pl.ds(col, 16)]
                q16 = v * 16 + lax.iota(jnp.int32, 16)
                pos = jnp.zeros((16,), jnp.int32)
                for s in (16, 8, 4, 2, 1):
                    probe = pos + (s - 1)
                    vals = plsc.load_gather(rows_v, [q16, probe])
                    pos = pos + jnp.where(vals < u16, s, 0)
                idx = c_v[pl.ds(v * 16, 16)] * 32 + pos
                at_v[h + r, pl.ds(col, 16)] = idx
                at_idx_v[pl.ds(v * 16, 16)] = idx
            return 0

        lax.fori_loop(0, 8, vecb, 0)

        pltpu.async_copy(xit1.at[at_idx_v], xg_v, sem_g).wait()

        def vecc(i, _):
            for k in range(4):
                v = i * 4 + k
                r = h + v // 8
                col = (v % 8) * 16
                xg = xg_v[pl.ds(v * 16, 16)]
                nz = noise_v[r, pl.ds(col, 16)]
                yt = yt_v[...]
                xt = PHI * xg + SIGMA_X * nz
                z = (yt - xt) * 2.0
                lw = -0.5 * (z * z) - _C1 - _C2
                xt_v[r, pl.ds(col, 16)] = xt
                lw_v[r, pl.ds(col, 16)] = lw
            return 0

        lax.fori_loop(0, 8, vecc, 0)

        @pl.when(b > 0)
        def _():
            pltpu.make_async_copy(xt_o.at[pl.ds(0, 4)],
                                  xt_v.at[pl.ds(hn, 4)], sem_out).wait()
            pltpu.make_async_copy(lw_o.at[pl.ds(0, 4)],
                                  lw_v.at[pl.ds(hn, 4)], sem_out).wait()
            pltpu.make_async_copy(at_o.at[pl.ds(0, 4)],
                                  at_v.at[pl.ds(hn, 4)], sem_out).wait()

        pltpu.async_copy(xt_v.at[pl.ds(h, 4)], xt_o.at[pl.ds(row0, 4)],
                         sem_out)
        pltpu.async_copy(lw_v.at[pl.ds(h, 4)], lw_o.at[pl.ds(row0, 4)],
                         sem_out)
        pltpu.async_copy(at_v.at[pl.ds(h, 4)], at_o.at[pl.ds(row0, 4)],
                         sem_out)
        return 0

    lax.fori_loop(0, _NB, batch, 0)

    hl = ((_NB - 1) % 2) * 4
    pltpu.make_async_copy(xt_o.at[pl.ds(0, 4)],
                          xt_v.at[pl.ds(hl, 4)], sem_out).wait()
    pltpu.make_async_copy(lw_o.at[pl.ds(0, 4)],
                          lw_v.at[pl.ds(hl, 4)], sem_out).wait()
    pltpu.make_async_copy(at_o.at[pl.ds(0, 4)],
                          at_v.at[pl.ds(hl, 4)], sem_out).wait()


@functools.cache
def _make_sc_search():
  return functools.partial(
    pl.kernel,
    mesh=plsc.VectorSubcoreMesh(core_axis_name="c", subcore_axis_name="s"),
    out_type=(jax.ShapeDtypeStruct((16384, 128), jnp.float32),
              jax.ShapeDtypeStruct((16384, 128), jnp.float32),
              jax.ShapeDtypeStruct((16384, 128), jnp.int32)),
    scratch_types=[pltpu.VMEM((128, 4, 128), jnp.float32),
                   pltpu.VMEM((8, 128), jnp.float32),
                   pltpu.VMEM((8, 128), jnp.float32),
                   pltpu.VMEM((512,), jnp.int32),
                   pltpu.VMEM((512, 32), jnp.float32),
                   pltpu.VMEM((8, 128), jnp.int32),
                   pltpu.VMEM((512,), jnp.int32),
                   pltpu.VMEM((512,), jnp.float32),
                   pltpu.VMEM((8, 128), jnp.float32),
                   pltpu.VMEM((8, 128), jnp.float32),
                   pltpu.VMEM((16,), jnp.float32),
                   pltpu.SemaphoreType.DMA,
                   pltpu.SemaphoreType.DMA,
                   pltpu.SemaphoreType.DMA],
    compiler_params=pltpu.CompilerParams(needs_layout_passes=False,
                                         use_tc_tiling_on_sc=False),
  )(_sc_search_body)


def kernel(xit_1, wt_1, yt, noise, uniforms):
    w3 = wt_1.reshape(16, 1024, 128)
    cw2d, cw32 = _tc_cdf(w3)
    xt2d, lw2d, at2d = _make_sc_search()(
        cw2d.reshape(65536, 32), cw32, uniforms.reshape(16384, 128),
        xit_1.reshape(N), noise.reshape(16384, 128),
        jnp.broadcast_to(yt.reshape(1), (16,)))
    return (xt2d.reshape(N, 1), lw2d.reshape(N, 1), at2d.reshape(N))


# speedup vs baseline: 1.1819x; 1.0478x over previous
import functools

import numpy as np
import jax
import jax.numpy as jnp
from jax import lax
from jax.experimental import pallas as pl
from jax.experimental.pallas import tpu as pltpu
from jax.experimental.pallas import tpu_sc as plsc

N = 2097152
PHI = 0.95
SIGMA_X = 0.3
SIGMA_Y = 0.5


def _tc_body(w3_ref, cw_ref, cw32_ref, pt_s, wnT_s, i1t_s, t1_s):
    f32 = jnp.float32
    lane = lax.broadcasted_iota(jnp.int32, (1, 128), 1)
    lane2 = lax.broadcasted_iota(jnp.int32, (128, 128), 1)
    row2 = lax.broadcasted_iota(jnp.int32, (128, 128), 0)

    rows16 = lax.broadcasted_iota(jnp.int32, (16, 128), 0)

    def ibody(i, acc):
        return acc + w3_ref[:, pl.ds(i, 1), :].reshape(16, 128)

    acc = lax.fori_loop(0, 1024, ibody, jnp.zeros((16, 128), f32))

    def gbody(g, s128):
        rowg = jnp.sum(jnp.where(rows16 == g, acc, 0.0), axis=0, keepdims=True)
        return s128 + rowg

    s128 = jnp.sum(jnp.where(rows16 == 0, acc, 0.0), axis=0, keepdims=True)
    s128 = lax.fori_loop(1, 16, gbody, s128)
    l1 = s128 + jnp.roll(s128, -1, axis=1)
    l2 = l1 + jnp.roll(l1, -2, axis=1)
    l3 = l2 + jnp.roll(l2, -4, axis=1)

    def jbody(j, s):
        return s + jnp.sum(jnp.where(lane == 8 * j, l3, 0.0))

    s_tot = lax.fori_loop(1, 16, jbody, jnp.sum(jnp.where(lane == 0, l3, 0.0)))

    def abody(t, _):
        gh = t // 8
        gl = (t % 8) * 128
        w_tile = w3_ref[gh, pl.ds(gl, 128), :]
        wnT_s[...] = jnp.transpose(w_tile) / s_tot
        c0 = wnT_s[pl.ds(0, 1), :]
        pt_s[pl.ds(t * 128, 1), :] = c0

        def kbody(k, c):
            c = c + wnT_s[pl.ds(k, 1), :]
            pt_s[pl.ds(t * 128 + k, 1), :] = c
            return c

        c_last = lax.fori_loop(1, 128, kbody, c0)
        t1_s[pl.ds(t, 1), :] = c_last
        return 0

    lax.fori_loop(0, 128, abody, 0)

    wnT_s[...] = jnp.transpose(t1_s[...])
    c0 = wnT_s[pl.ds(0, 1), :]
    i1t_s[pl.ds(0, 1), :] = c0

    def k1body(k, c):
        c = c + wnT_s[pl.ds(k, 1), :]
        i1t_s[pl.ds(k, 1), :] = c
        return c

    t2 = lax.fori_loop(1, 128, k1body, c0)

    e0 = jnp.sum(jnp.where(lane == 0, t2, 0.0))
    c2_0 = jnp.where(lane == 0, e0, jnp.zeros((1, 128), f32))

    def fold2(j, carry):
        s, c2 = carry
        s = s + jnp.sum(jnp.where(lane == j, t2, 0.0))
        c2 = jnp.where(lane == j, s, c2)
        return (s, c2)

    _, c2 = lax.fori_loop(1, 128, fold2, (e0, c2_0))
    off1 = jnp.where(lane == 0, 0.0, jnp.roll(c2, 1, axis=1))

    outer1t = i1t_s[...] + off1
    outer1n = jnp.transpose(outer1t)
    rolled = jnp.roll(outer1n, 1, axis=1)
    rows_rolled = jnp.roll(outer1n, 1, axis=0)
    lastcol = jnp.sum(jnp.where(lane2 == 127, rows_rolled, 0.0), axis=1,
                      keepdims=True)
    off0 = jnp.where(lane2 == 0, lastcol, rolled)
    off0 = jnp.where((lane2 == 0) & (row2 == 0), 0.0, off0)
    wnT_s[...] = off0

    def cbody(t, _):
        off0row = wnT_s[pl.ds(t, 1), :]
        cwt = pt_s[pl.ds(t * 128, 128), :] + off0row
        cw_ref[pl.ds(t * 128, 128), :] = jnp.transpose(cwt)
        for q in range(4):
            rowq = jnp.sum(jnp.where(row2 == 32 * q + 31, cwt, 0.0), axis=0,
                           keepdims=True)
            cw32_ref[pl.ds(t, 1), pl.ds(q, 1), :] = rowq.reshape(1, 1, 128)
        return 0

    lax.fori_loop(0, 128, cbody, 0)


def _tc_cdf(w3):
    return pl.pallas_call(
        _tc_body,
        out_shape=(jax.ShapeDtypeStruct((16384, 128), jnp.float32),
                   jax.ShapeDtypeStruct((128, 4, 128), jnp.float32)),
        scratch_shapes=[pltpu.VMEM((16384, 128), jnp.float32),
                        pltpu.VMEM((128, 128), jnp.float32),
                        pltpu.VMEM((128, 128), jnp.float32),
                        pltpu.VMEM((128, 128), jnp.float32)],
    )(w3)


_NW = 32
_BQ = 1024
_NB = N // _NW // _BQ
_C1 = float(np.log(np.float32(SIGMA_Y)))
_C2 = float(np.float32(0.5) * np.log(np.float32(2.0) * np.pi))


def _sc_search_body(cwseg, cw32_h, u2d, xit1, noise2d, yt16,
                    xt_o, lw_o, at_o,
                    cw32_v, u_v, noise_v, c_v, rows_v, at_v, at_idx_v,
                    xg_v, xt_v, lw_v, yt_v, sem_g, sem_in, sem_out):
    wid = lax.axis_index("s") * 2 + lax.axis_index("c")
    pltpu.sync_copy(cw32_h, cw32_v)
    pltpu.sync_copy(yt16, yt_v)
    base_row = wid * (_NB * 8)

    pltpu.async_copy(u2d.at[pl.ds(base_row, 8)], u_v.at[pl.ds(0, 8)], sem_in)
    pltpu.async_copy(noise2d.at[pl.ds(base_row, 8)],
                     noise_v.at[pl.ds(0, 8)], sem_in)

    def batch(b, _):
        h = (b % 2) * 8
        hn = 8 - h
        row0 = base_row + b * 8

        pltpu.make_async_copy(u2d.at[pl.ds(0, 8)],
                              u_v.at[pl.ds(h, 8)], sem_in).wait()
        pltpu.make_async_copy(noise2d.at[pl.ds(0, 8)],
                              noise_v.at[pl.ds(h, 8)], sem_in).wait()

        def veca(i, _):
            for k in range(4):
                v = i * 4 + k
                r = v // 8
                col = (v % 8) * 16
                u16 = u_v[h + r, pl.ds(col, 16)]
                base = jnp.zeros((16,), jnp.int32)
                for s in (32768, 16384, 8192, 4096, 2048, 1024, 512, 256,
                          128, 64, 32, 16, 8, 4, 2, 1):
                    probe = base + (s - 1)
                    t16 = lax.shift_right_logical(probe, 9)
                    q16 = probe & 3
                    cc16 = lax.shift_right_logical(probe, 2) & 127
                    vals = plsc.load_gather(cw32_v, [t16, q16, cc16])
                    base = base + jnp.where(vals < u16, s, 0)
                c_v[pl.ds(v * 16, 16)] = base
            return 0

        lax.fori_loop(0, 16, veca, 0)

        h_rows = pltpu.async_copy(cwseg.at[c_v], rows_v, sem_g)

        @pl.when(b + 1 < _NB)
        def _():
            pltpu.async_copy(u2d.at[pl.ds(row0 + 8, 8)],
                             u_v.at[pl.ds(hn, 8)], sem_in)
            pltpu.async_copy(noise2d.at[pl.ds(row0 + 8, 8)],
                             noise_v.at[pl.ds(hn, 8)], sem_in)

        h_rows.wait()

        def vecb(i, _):
            for k in range(4):
                v = i * 4 + k
                r = v // 8
                col = (v % 8) * 16
                u16 = u_v[h + r, pl.ds(col, 16)]
                q16 = v * 16 + lax.iota(jnp.int32, 16)
                pos = jnp.zeros((16,), jnp.int32)
                for s in (16, 8, 4, 2, 1):
                    probe = pos + (s - 1)
                    vals = plsc.load_gather(rows_v, [q16, probe])
                    pos = pos + jnp.where(vals < u16, s, 0)
                idx = c_v[pl.ds(v * 16, 16)] * 32 + pos
                at_v[h + r, pl.ds(col, 16)] = idx
                at_idx_v[pl.ds(v * 16, 16)] = idx
            return 0

        lax.fori_loop(0, 16, vecb, 0)

        pltpu.async_copy(xit1.at[at_idx_v], xg_v, sem_g).wait()

        def vecc(i, _):
            for k in range(4):
                v = i * 4 + k
                r = h + v // 8
                col = (v % 8) * 16
                xg = xg_v[pl.ds(v * 16, 16)]
                nz = noise_v[r, pl.ds(col, 16)]
                yt = yt_v[...]
                xt = PHI * xg + SIGMA_X * nz
                z = (yt - xt) * 2.0
                lw = -0.5 * (z * z) - _C1 - _C2
                xt_v[r, pl.ds(col, 16)] = xt
                lw_v[r, pl.ds(col, 16)] = lw
            return 0

        lax.fori_loop(0, 16, vecc, 0)

        @pl.when(b > 0)
        def _():
            pltpu.make_async_copy(xt_o.at[pl.ds(0, 8)],
                                  xt_v.at[pl.ds(hn, 8)], sem_out).wait()
            pltpu.make_async_copy(lw_o.at[pl.ds(0, 8)],
                                  lw_v.at[pl.ds(hn, 8)], sem_out).wait()
            pltpu.make_async_copy(at_o.at[pl.ds(0, 8)],
                                  at_v.at[pl.ds(hn, 8)], sem_out).wait()

        pltpu.async_copy(xt_v.at[pl.ds(h, 8)], xt_o.at[pl.ds(row0, 8)],
                         sem_out)
        pltpu.async_copy(lw_v.at[pl.ds(h, 8)], lw_o.at[pl.ds(row0, 8)],
                         sem_out)
        pltpu.async_copy(at_v.at[pl.ds(h, 8)], at_o.at[pl.ds(row0, 8)],
                         sem_out)
        return 0

    lax.fori_loop(0, _NB, batch, 0)

    hl = ((_NB - 1) % 2) * 8
    pltpu.make_async_copy(xt_o.at[pl.ds(0, 8)],
                          xt_v.at[pl.ds(hl, 8)], sem_out).wait()
    pltpu.make_async_copy(lw_o.at[pl.ds(0, 8)],
                          lw_v.at[pl.ds(hl, 8)], sem_out).wait()
    pltpu.make_async_copy(at_o.at[pl.ds(0, 8)],
                          at_v.at[pl.ds(hl, 8)], sem_out).wait()


@functools.cache
def _make_sc_search():
  return functools.partial(
    pl.kernel,
    mesh=plsc.VectorSubcoreMesh(core_axis_name="c", subcore_axis_name="s"),
    out_type=(jax.ShapeDtypeStruct((16384, 128), jnp.float32),
              jax.ShapeDtypeStruct((16384, 128), jnp.float32),
              jax.ShapeDtypeStruct((16384, 128), jnp.int32)),
    scratch_types=[pltpu.VMEM((128, 4, 128), jnp.float32),
                   pltpu.VMEM((16, 128), jnp.float32),
                   pltpu.VMEM((16, 128), jnp.float32),
                   pltpu.VMEM((1024,), jnp.int32),
                   pltpu.VMEM((1024, 32), jnp.float32),
                   pltpu.VMEM((16, 128), jnp.int32),
                   pltpu.VMEM((1024,), jnp.int32),
                   pltpu.VMEM((1024,), jnp.float32),
                   pltpu.VMEM((16, 128), jnp.float32),
                   pltpu.VMEM((16, 128), jnp.float32),
                   pltpu.VMEM((16,), jnp.float32),
                   pltpu.SemaphoreType.DMA,
                   pltpu.SemaphoreType.DMA,
                   pltpu.SemaphoreType.DMA],
    compiler_params=pltpu.CompilerParams(needs_layout_passes=False,
                                         use_tc_tiling_on_sc=False),
  )(_sc_search_body)


def kernel(xit_1, wt_1, yt, noise, uniforms):
    w3 = wt_1.reshape(16, 1024, 128)
    cw2d, cw32 = _tc_cdf(w3)
    xt2d, lw2d, at2d = _make_sc_search()(
        cw2d.reshape(65536, 32), cw32, uniforms.reshape(16384, 128),
        xit_1.reshape(N), noise.reshape(16384, 128),
        jnp.broadcast_to(yt.reshape(1), (16,)))
    return (xt2d.reshape(N, 1), lw2d.reshape(N, 1), at2d.reshape(N))


# speedup vs baseline: 1.7017x; 1.4398x over previous
import functools

import numpy as np
import jax
import jax.numpy as jnp
from jax import lax
from jax.experimental import pallas as pl
from jax.experimental.pallas import tpu as pltpu
from jax.experimental.pallas import tpu_sc as plsc

N = 2097152
PHI = 0.95
SIGMA_X = 0.3
SIGMA_Y = 0.5


def _tc_body(w3_ref, cw_ref, cw32_ref, pt_s, wnT_s, i1t_s, t1_s):
    f32 = jnp.float32
    lane = lax.broadcasted_iota(jnp.int32, (1, 128), 1)
    lane2 = lax.broadcasted_iota(jnp.int32, (128, 128), 1)
    row2 = lax.broadcasted_iota(jnp.int32, (128, 128), 0)

    rows16 = lax.broadcasted_iota(jnp.int32, (16, 128), 0)

    def ibody(i, acc):
        return acc + w3_ref[:, pl.ds(i, 1), :].reshape(16, 128)

    acc = lax.fori_loop(0, 1024, ibody, jnp.zeros((16, 128), f32))

    def gbody(g, s128):
        rowg = jnp.sum(jnp.where(rows16 == g, acc, 0.0), axis=0, keepdims=True)
        return s128 + rowg

    s128 = jnp.sum(jnp.where(rows16 == 0, acc, 0.0), axis=0, keepdims=True)
    s128 = lax.fori_loop(1, 16, gbody, s128)
    l1 = s128 + jnp.roll(s128, -1, axis=1)
    l2 = l1 + jnp.roll(l1, -2, axis=1)
    l3 = l2 + jnp.roll(l2, -4, axis=1)

    def jbody(j, s):
        return s + jnp.sum(jnp.where(lane == 8 * j, l3, 0.0))

    s_tot = lax.fori_loop(1, 16, jbody, jnp.sum(jnp.where(lane == 0, l3, 0.0)))

    def abody(t, _):
        gh = t // 8
        gl = (t % 8) * 128
        w_tile = w3_ref[gh, pl.ds(gl, 128), :]
        wnT_s[...] = jnp.transpose(w_tile) / s_tot
        c0 = wnT_s[pl.ds(0, 1), :]
        pt_s[pl.ds(t * 128, 1), :] = c0

        def kbody(k, c):
            c = c + wnT_s[pl.ds(k, 1), :]
            pt_s[pl.ds(t * 128 + k, 1), :] = c
            return c

        c_last = lax.fori_loop(1, 128, kbody, c0)
        t1_s[pl.ds(t, 1), :] = c_last
        return 0

    lax.fori_loop(0, 128, abody, 0)

    wnT_s[...] = jnp.transpose(t1_s[...])
    c0 = wnT_s[pl.ds(0, 1), :]
    i1t_s[pl.ds(0, 1), :] = c0

    def k1body(k, c):
        c = c + wnT_s[pl.ds(k, 1), :]
        i1t_s[pl.ds(k, 1), :] = c
        return c

    t2 = lax.fori_loop(1, 128, k1body, c0)

    e0 = jnp.sum(jnp.where(lane == 0, t2, 0.0))
    c2_0 = jnp.where(lane == 0, e0, jnp.zeros((1, 128), f32))

    def fold2(j, carry):
        s, c2 = carry
        s = s + jnp.sum(jnp.where(lane == j, t2, 0.0))
        c2 = jnp.where(lane == j, s, c2)
        return (s, c2)

    _, c2 = lax.fori_loop(1, 128, fold2, (e0, c2_0))
    off1 = jnp.where(lane == 0, 0.0, jnp.roll(c2, 1, axis=1))

    outer1t = i1t_s[...] + off1
    outer1n = jnp.transpose(outer1t)
    rolled = jnp.roll(outer1n, 1, axis=1)
    rows_rolled = jnp.roll(outer1n, 1, axis=0)
    lastcol = jnp.sum(jnp.where(lane2 == 127, rows_rolled, 0.0), axis=1,
                      keepdims=True)
    off0 = jnp.where(lane2 == 0, lastcol, rolled)
    off0 = jnp.where((lane2 == 0) & (row2 == 0), 0.0, off0)
    wnT_s[...] = off0

    def cbody(t, _):
        off0row = wnT_s[pl.ds(t, 1), :]
        cwt = pt_s[pl.ds(t * 128, 128), :] + off0row
        cw_ref[pl.ds(t * 128, 128), :] = jnp.transpose(cwt)
        for q in range(4):
            rowq = jnp.sum(jnp.where(row2 == 32 * q + 31, cwt, 0.0), axis=0,
                           keepdims=True)
            cw32_ref[pl.ds(t, 1), pl.ds(q, 1), :] = rowq.reshape(1, 1, 128)
        return 0

    lax.fori_loop(0, 128, cbody, 0)


def _tc_cdf(w3):
    return pl.pallas_call(
        _tc_body,
        out_shape=(jax.ShapeDtypeStruct((16384, 128), jnp.float32),
                   jax.ShapeDtypeStruct((128, 4, 128), jnp.float32)),
        scratch_shapes=[pltpu.VMEM((16384, 128), jnp.float32),
                        pltpu.VMEM((128, 128), jnp.float32),
                        pltpu.VMEM((128, 128), jnp.float32),
                        pltpu.VMEM((128, 128), jnp.float32)],
    )(w3)


_NW = 32
_BQ = 1024
_NB = N // _NW // _BQ
_C1 = float(np.log(np.float32(SIGMA_Y)))
_C2 = float(np.float32(0.5) * np.log(np.float32(2.0) * np.pi))


def _sc_search_body(cwseg, cw32_h, u2d, xit1, noise2d, yt16,
                    xt_o, lw_o, at_o,
                    cw32_v, u_v, noise_v, c_v, rows_v, at_v, at_idx_v,
                    xg_v, xt_v, lw_v, yt_v, sem_g, sem_in, sem_out):
    wid = lax.axis_index("s") * 2 + lax.axis_index("c")
    pltpu.sync_copy(cw32_h, cw32_v)
    pltpu.sync_copy(yt16, yt_v)
    base_row = wid * (_NB * 8)

    pltpu.async_copy(u2d.at[pl.ds(base_row, 8)], u_v.at[pl.ds(0, 8)], sem_in)
    pltpu.async_copy(noise2d.at[pl.ds(base_row, 8)],
                     noise_v.at[pl.ds(0, 8)], sem_in)

    def batch(b, _):
        h = (b % 2) * 8
        hn = 8 - h
        row0 = base_row + b * 8

        pltpu.make_async_copy(u2d.at[pl.ds(0, 8)],
                              u_v.at[pl.ds(h, 8)], sem_in).wait()
        pltpu.make_async_copy(noise2d.at[pl.ds(0, 8)],
                              noise_v.at[pl.ds(h, 8)], sem_in).wait()

        def veca(i, _):
            us = [u_v[h + (i * 4 + k) // 8, pl.ds(((i * 4 + k) % 8) * 16, 16)]
                  for k in range(4)]
            bases = [jnp.zeros((16,), jnp.int32) for _ in range(4)]
            for s in (32768, 16384, 8192, 4096, 2048, 1024, 512, 256,
                      128, 64, 32, 16, 8, 4, 2, 1):
                probes = [bases[k] + (s - 1) for k in range(4)]
                vals = [plsc.load_gather(
                            cw32_v,
                            [lax.shift_right_logical(probes[k], 9),
                             probes[k] & 3,
                             lax.shift_right_logical(probes[k], 2) & 127])
                        for k in range(4)]
                bases = [bases[k] + jnp.where(vals[k] < us[k], s, 0)
                         for k in range(4)]
            for k in range(4):
                c_v[pl.ds((i * 4 + k) * 16, 16)] = bases[k]
            return 0

        lax.fori_loop(0, 16, veca, 0)

        h_rows = pltpu.async_copy(cwseg.at[c_v], rows_v, sem_g)

        @pl.when(b + 1 < _NB)
        def _():
            pltpu.async_copy(u2d.at[pl.ds(row0 + 8, 8)],
                             u_v.at[pl.ds(hn, 8)], sem_in)
            pltpu.async_copy(noise2d.at[pl.ds(row0 + 8, 8)],
                             noise_v.at[pl.ds(hn, 8)], sem_in)

        h_rows.wait()

        def vecb(i, _):
            us = [u_v[h + (i * 4 + k) // 8, pl.ds(((i * 4 + k) % 8) * 16, 16)]
                  for k in range(4)]
            qs = [(i * 4 + k) * 16 + lax.iota(jnp.int32, 16) for k in range(4)]
            poss = [jnp.zeros((16,), jnp.int32) for _ in range(4)]
            for s in (16, 8, 4, 2, 1):
                vals = [plsc.load_gather(rows_v, [qs[k], poss[k] + (s - 1)])
                        for k in range(4)]
                poss = [poss[k] + jnp.where(vals[k] < us[k], s, 0)
                        for k in range(4)]
            for k in range(4):
                v = i * 4 + k
                idx = c_v[pl.ds(v * 16, 16)] * 32 + poss[k]
                at_v[h + v // 8, pl.ds((v % 8) * 16, 16)] = idx
                at_idx_v[pl.ds(v * 16, 16)] = idx
            return 0

        lax.fori_loop(0, 16, vecb, 0)

        pltpu.async_copy(xit1.at[at_idx_v], xg_v, sem_g).wait()

        def vecc(i, _):
            for k in range(4):
                v = i * 4 + k
                r = h + v // 8
                col = (v % 8) * 16
                xg = xg_v[pl.ds(v * 16, 16)]
                nz = noise_v[r, pl.ds(col, 16)]
                yt = yt_v[...]
                xt = PHI * xg + SIGMA_X * nz
                z = (yt - xt) * 2.0
                lw = -0.5 * (z * z) - _C1 - _C2
                xt_v[r, pl.ds(col, 16)] = xt
                lw_v[r, pl.ds(col, 16)] = lw
            return 0

        lax.fori_loop(0, 16, vecc, 0)

        @pl.when(b > 0)
        def _():
            pltpu.make_async_copy(xt_o.at[pl.ds(0, 8)],
                                  xt_v.at[pl.ds(hn, 8)], sem_out).wait()
            pltpu.make_async_copy(lw_o.at[pl.ds(0, 8)],
                                  lw_v.at[pl.ds(hn, 8)], sem_out).wait()
            pltpu.make_async_copy(at_o.at[pl.ds(0, 8)],
                                  at_v.at[pl.ds(hn, 8)], sem_out).wait()

        pltpu.async_copy(xt_v.at[pl.ds(h, 8)], xt_o.at[pl.ds(row0, 8)],
                         sem_out)
        pltpu.async_copy(lw_v.at[pl.ds(h, 8)], lw_o.at[pl.ds(row0, 8)],
                         sem_out)
        pltpu.async_copy(at_v.at[pl.ds(h, 8)], at_o.at[pl.ds(row0, 8)],
                         sem_out)
        return 0

    lax.fori_loop(0, _NB, batch, 0)

    hl = ((_NB - 1) % 2) * 8
    pltpu.make_async_copy(xt_o.at[pl.ds(0, 8)],
                          xt_v.at[pl.ds(hl, 8)], sem_out).wait()
    pltpu.make_async_copy(lw_o.at[pl.ds(0, 8)],
                          lw_v.at[pl.ds(hl, 8)], sem_out).wait()
    pltpu.make_async_copy(at_o.at[pl.ds(0, 8)],
                          at_v.at[pl.ds(hl, 8)], sem_out).wait()


@functools.cache
def _make_sc_search():
  return functools.partial(
    pl.kernel,
    mesh=plsc.VectorSubcoreMesh(core_axis_name="c", subcore_axis_name="s"),
    out_type=(jax.ShapeDtypeStruct((16384, 128), jnp.float32),
              jax.ShapeDtypeStruct((16384, 128), jnp.float32),
              jax.ShapeDtypeStruct((16384, 128), jnp.int32)),
    scratch_types=[pltpu.VMEM((128, 4, 128), jnp.float32),
                   pltpu.VMEM((16, 128), jnp.float32),
                   pltpu.VMEM((16, 128), jnp.float32),
                   pltpu.VMEM((1024,), jnp.int32),
                   pltpu.VMEM((1024, 32), jnp.float32),
                   pltpu.VMEM((16, 128), jnp.int32),
                   pltpu.VMEM((1024,), jnp.int32),
                   pltpu.VMEM((1024,), jnp.float32),
                   pltpu.VMEM((16, 128), jnp.float32),
                   pltpu.VMEM((16, 128), jnp.float32),
                   pltpu.VMEM((16,), jnp.float32),
                   pltpu.SemaphoreType.DMA,
                   pltpu.SemaphoreType.DMA,
                   pltpu.SemaphoreType.DMA],
    compiler_params=pltpu.CompilerParams(needs_layout_passes=False,
                                         use_tc_tiling_on_sc=False),
  )(_sc_search_body)


def kernel(xit_1, wt_1, yt, noise, uniforms):
    w3 = wt_1.reshape(16, 1024, 128)
    cw2d, cw32 = _tc_cdf(w3)
    xt2d, lw2d, at2d = _make_sc_search()(
        cw2d.reshape(65536, 32), cw32, uniforms.reshape(16384, 128),
        xit_1.reshape(N), noise.reshape(16384, 128),
        jnp.broadcast_to(yt.reshape(1), (16,)))
    return (xt2d.reshape(N, 1), lw2d.reshape(N, 1), at2d.reshape(N))


# speedup vs baseline: 1.8541x; 1.0896x over previous
import functools

import numpy as np
import jax
import jax.numpy as jnp
from jax import lax
from jax.experimental import pallas as pl
from jax.experimental.pallas import tpu as pltpu
from jax.experimental.pallas import tpu_sc as plsc

N = 2097152
PHI = 0.95
SIGMA_X = 0.3
SIGMA_Y = 0.5


def _tc_body(w3_ref, cw_ref, cw32_ref, pt_s, wnT_s, i1t_s, t1_s):
    f32 = jnp.float32
    lane = lax.broadcasted_iota(jnp.int32, (1, 128), 1)
    lane2 = lax.broadcasted_iota(jnp.int32, (128, 128), 1)
    row2 = lax.broadcasted_iota(jnp.int32, (128, 128), 0)

    rows16 = lax.broadcasted_iota(jnp.int32, (16, 128), 0)

    def ibody(i, acc):
        return acc + w3_ref[:, pl.ds(i, 1), :].reshape(16, 128)

    acc = lax.fori_loop(0, 1024, ibody, jnp.zeros((16, 128), f32))

    def gbody(g, s128):
        rowg = jnp.sum(jnp.where(rows16 == g, acc, 0.0), axis=0, keepdims=True)
        return s128 + rowg

    s128 = jnp.sum(jnp.where(rows16 == 0, acc, 0.0), axis=0, keepdims=True)
    s128 = lax.fori_loop(1, 16, gbody, s128)
    l1 = s128 + jnp.roll(s128, -1, axis=1)
    l2 = l1 + jnp.roll(l1, -2, axis=1)
    l3 = l2 + jnp.roll(l2, -4, axis=1)

    def jbody(j, s):
        return s + jnp.sum(jnp.where(lane == 8 * j, l3, 0.0))

    s_tot = lax.fori_loop(1, 16, jbody, jnp.sum(jnp.where(lane == 0, l3, 0.0)))

    def abody(t, _):
        gh = t // 8
        gl = (t % 8) * 128
        w_tile = w3_ref[gh, pl.ds(gl, 128), :]
        wnT_s[...] = jnp.transpose(w_tile) / s_tot
        c0 = wnT_s[pl.ds(0, 1), :]
        pt_s[pl.ds(t * 128, 1), :] = c0

        def kbody(k, c):
            c = c + wnT_s[pl.ds(k, 1), :]
            pt_s[pl.ds(t * 128 + k, 1), :] = c
            return c

        c_last = lax.fori_loop(1, 128, kbody, c0)
        t1_s[pl.ds(t, 1), :] = c_last
        return 0

    lax.fori_loop(0, 128, abody, 0)

    wnT_s[...] = jnp.transpose(t1_s[...])
    c0 = wnT_s[pl.ds(0, 1), :]
    i1t_s[pl.ds(0, 1), :] = c0

    def k1body(k, c):
        c = c + wnT_s[pl.ds(k, 1), :]
        i1t_s[pl.ds(k, 1), :] = c
        return c

    t2 = lax.fori_loop(1, 128, k1body, c0)

    e0 = jnp.sum(jnp.where(lane == 0, t2, 0.0))
    c2_0 = jnp.where(lane == 0, e0, jnp.zeros((1, 128), f32))

    def fold2(j, carry):
        s, c2 = carry
        s = s + jnp.sum(jnp.where(lane == j, t2, 0.0))
        c2 = jnp.where(lane == j, s, c2)
        return (s, c2)

    _, c2 = lax.fori_loop(1, 128, fold2, (e0, c2_0))
    off1 = jnp.where(lane == 0, 0.0, jnp.roll(c2, 1, axis=1))

    outer1t = i1t_s[...] + off1
    outer1n = jnp.transpose(outer1t)
    rolled = jnp.roll(outer1n, 1, axis=1)
    rows_rolled = jnp.roll(outer1n, 1, axis=0)
    lastcol = jnp.sum(jnp.where(lane2 == 127, rows_rolled, 0.0), axis=1,
                      keepdims=True)
    off0 = jnp.where(lane2 == 0, lastcol, rolled)
    off0 = jnp.where((lane2 == 0) & (row2 == 0), 0.0, off0)
    wnT_s[...] = off0

    def cbody(t, _):
        off0row = wnT_s[pl.ds(t, 1), :]
        cwt = pt_s[pl.ds(t * 128, 128), :] + off0row
        cw_ref[pl.ds(t * 128, 128), :] = jnp.transpose(cwt)
        for q in range(4):
            rowq = jnp.sum(jnp.where(row2 == 32 * q + 31, cwt, 0.0), axis=0,
                           keepdims=True)
            cw32_ref[pl.ds(t, 1), pl.ds(q, 1), :] = rowq.reshape(1, 1, 128)
        return 0

    lax.fori_loop(0, 128, cbody, 0)


def _tc_cdf(w3):
    return pl.pallas_call(
        _tc_body,
        out_shape=(jax.ShapeDtypeStruct((16384, 128), jnp.float32),
                   jax.ShapeDtypeStruct((128, 4, 128), jnp.float32)),
        scratch_shapes=[pltpu.VMEM((16384, 128), jnp.float32),
                        pltpu.VMEM((128, 128), jnp.float32),
                        pltpu.VMEM((128, 128), jnp.float32),
                        pltpu.VMEM((128, 128), jnp.float32)],
    )(w3)


_NW = 32
_BQ = 1024
_NB = N // _NW // _BQ
_C1 = float(np.log(np.float32(SIGMA_Y)))
_C2 = float(np.float32(0.5) * np.log(np.float32(2.0) * np.pi))


def _sc_search_body(cwseg, cw32_h, u2d, xit1, noise2d, yt16,
                    xt_o, lw_o, at_o,
                    cw32_v, u_v, noise_v, c_v, rows_v, at_v, at_idx_v,
                    xg_v, xt_v, lw_v, yt_v, sem_g, sem_in, sem_out):
    wid = lax.axis_index("s") * 2 + lax.axis_index("c")
    pltpu.sync_copy(cw32_h, cw32_v)
    pltpu.sync_copy(yt16, yt_v)
    base_row = wid * (_NB * 8)

    pltpu.async_copy(u2d.at[pl.ds(base_row, 8)], u_v.at[pl.ds(0, 8)], sem_in)
    pltpu.async_copy(noise2d.at[pl.ds(base_row, 8)],
                     noise_v.at[pl.ds(0, 8)], sem_in)

    def batch(b, _):
        h = (b % 2) * 8
        hn = 8 - h
        row0 = base_row + b * 8

        pltpu.make_async_copy(u2d.at[pl.ds(0, 8)],
                              u_v.at[pl.ds(h, 8)], sem_in).wait()
        pltpu.make_async_copy(noise2d.at[pl.ds(0, 8)],
                              noise_v.at[pl.ds(h, 8)], sem_in).wait()

        def veca(i, _):
            us = [u_v[h + (i * 8 + k) // 8, pl.ds(((i * 8 + k) % 8) * 16, 16)]
                  for k in range(8)]
            bases = [jnp.zeros((16,), jnp.int32) for _ in range(8)]
            for s in (32768, 16384, 8192, 4096, 2048, 1024, 512, 256,
                      128, 64, 32, 16, 8, 4, 2, 1):
                probes = [bases[k] + (s - 1) for k in range(8)]
                vals = [plsc.load_gather(
                            cw32_v,
                            [lax.shift_right_logical(probes[k], 9),
                             probes[k] & 3,
                             lax.shift_right_logical(probes[k], 2) & 127])
                        for k in range(8)]
                bases = [bases[k] + jnp.where(vals[k] < us[k], s, 0)
                         for k in range(8)]
            for k in range(8):
                c_v[pl.ds((i * 8 + k) * 16, 16)] = bases[k]
            return 0

        lax.fori_loop(0, 8, veca, 0)

        h_rows = pltpu.async_copy(cwseg.at[c_v], rows_v, sem_g)

        @pl.when(b + 1 < _NB)
        def _():
            pltpu.async_copy(u2d.at[pl.ds(row0 + 8, 8)],
                             u_v.at[pl.ds(hn, 8)], sem_in)
            pltpu.async_copy(noise2d.at[pl.ds(row0 + 8, 8)],
                             noise_v.at[pl.ds(hn, 8)], sem_in)

        h_rows.wait()

        def vecb(i, _):
            us = [u_v[h + (i * 8 + k) // 8, pl.ds(((i * 8 + k) % 8) * 16, 16)]
                  for k in range(8)]
            qs = [(i * 8 + k) * 16 + lax.iota(jnp.int32, 16) for k in range(8)]
            poss = [jnp.zeros((16,), jnp.int32) for _ in range(8)]
            for s in (16, 8, 4, 2, 1):
                vals = [plsc.load_gather(rows_v, [qs[k], poss[k] + (s - 1)])
                        for k in range(8)]
                poss = [poss[k] + jnp.where(vals[k] < us[k], s, 0)
                        for k in range(8)]
            for k in range(8):
                v = i * 8 + k
                idx = c_v[pl.ds(v * 16, 16)] * 32 + poss[k]
                at_v[h + v // 8, pl.ds((v % 8) * 16, 16)] = idx
                at_idx_v[pl.ds(v * 16, 16)] = idx
            return 0

        lax.fori_loop(0, 8, vecb, 0)

        pltpu.async_copy(xit1.at[at_idx_v], xg_v, sem_g).wait()

        def vecc(i, _):
            for k in range(4):
                v = i * 4 + k
                r = h + v // 8
                col = (v % 8) * 16
                xg = xg_v[pl.ds(v * 16, 16)]
                nz = noise_v[r, pl.ds(col, 16)]
                yt = yt_v[...]
                xt = PHI * xg + SIGMA_X * nz
                z = (yt - xt) * 2.0
                lw = -0.5 * (z * z) - _C1 - _C2
                xt_v[r, pl.ds(col, 16)] = xt
                lw_v[r, pl.ds(col, 16)] = lw
            return 0

        lax.fori_loop(0, 16, vecc, 0)

        @pl.when(b > 0)
        def _():
            pltpu.make_async_copy(xt_o.at[pl.ds(0, 8)],
                                  xt_v.at[pl.ds(hn, 8)], sem_out).wait()
            pltpu.make_async_copy(lw_o.at[pl.ds(0, 8)],
                                  lw_v.at[pl.ds(hn, 8)], sem_out).wait()
            pltpu.make_async_copy(at_o.at[pl.ds(0, 8)],
                                  at_v.at[pl.ds(hn, 8)], sem_out).wait()

        pltpu.async_copy(xt_v.at[pl.ds(h, 8)], xt_o.at[pl.ds(row0, 8)],
                         sem_out)
        pltpu.async_copy(lw_v.at[pl.ds(h, 8)], lw_o.at[pl.ds(row0, 8)],
                         sem_out)
        pltpu.async_copy(at_v.at[pl.ds(h, 8)], at_o.at[pl.ds(row0, 8)],
                         sem_out)
        return 0

    lax.fori_loop(0, _NB, batch, 0)

    hl = ((_NB - 1) % 2) * 8
    pltpu.make_async_copy(xt_o.at[pl.ds(0, 8)],
                          xt_v.at[pl.ds(hl, 8)], sem_out).wait()
    pltpu.make_async_copy(lw_o.at[pl.ds(0, 8)],
                          lw_v.at[pl.ds(hl, 8)], sem_out).wait()
    pltpu.make_async_copy(at_o.at[pl.ds(0, 8)],
                          at_v.at[pl.ds(hl, 8)], sem_out).wait()


@functools.cache
def _make_sc_search():
  return functools.partial(
    pl.kernel,
    mesh=plsc.VectorSubcoreMesh(core_axis_name="c", subcore_axis_name="s"),
    out_type=(jax.ShapeDtypeStruct((16384, 128), jnp.float32),
              jax.ShapeDtypeStruct((16384, 128), jnp.float32),
              jax.ShapeDtypeStruct((16384, 128), jnp.int32)),
    scratch_types=[pltpu.VMEM((128, 4, 128), jnp.float32),
                   pltpu.VMEM((16, 128), jnp.float32),
                   pltpu.VMEM((16, 128), jnp.float32),
                   pltpu.VMEM((1024,), jnp.int32),
                   pltpu.VMEM((1024, 32), jnp.float32),
                   pltpu.VMEM((16, 128), jnp.int32),
                   pltpu.VMEM((1024,), jnp.int32),
                   pltpu.VMEM((1024,), jnp.float32),
                   pltpu.VMEM((16, 128), jnp.float32),
                   pltpu.VMEM((16, 128), jnp.float32),
                   pltpu.VMEM((16,), jnp.float32),
                   pltpu.SemaphoreType.DMA,
                   pltpu.SemaphoreType.DMA,
                   pltpu.SemaphoreType.DMA],
    compiler_params=pltpu.CompilerParams(needs_layout_passes=False,
                                         use_tc_tiling_on_sc=False),
  )(_sc_search_body)


def kernel(xit_1, wt_1, yt, noise, uniforms):
    w3 = wt_1.reshape(16, 1024, 128)
    cw2d, cw32 = _tc_cdf(w3)
    xt2d, lw2d, at2d = _make_sc_search()(
        cw2d.reshape(65536, 32), cw32, uniforms.reshape(16384, 128),
        xit_1.reshape(N), noise.reshape(16384, 128),
        jnp.broadcast_to(yt.reshape(1), (16,)))
    return (xt2d.reshape(N, 1), lw2d.reshape(N, 1), at2d.reshape(N))
